# Initial kernel scaffold; baseline (speedup 1.0000x reference)
import jax
import jax.numpy as jnp
from jax.experimental import pallas as pl


def kernel(V_obs, A_obs, Wg0, bg0, Wt0, bt0, Wg1, bg1, Wt1, bt1, Wg2, bg2, Wt2, bt2, Wg3, bg3, Wt3, bt3, Wg4, bg4, Wt4, bt4, Wtime):
    raise NotImplementedError("write your pallas kernel here")


# speedup vs baseline: 60.5792x; 60.5792x over previous
import jax
import jax.numpy as jnp
from jax.experimental import pallas as pl
from jax.experimental.pallas import tpu as pltpu

_R = 3
_SCALES = (0.5, 1.0, 2.0)
_GROUP_TH = 2.0
_T = 8
_N = 256
_PRED = 12
_C = 64

_DN_T = (((1,), (1,)), ((), ()))
_CINS = (2, _C, _C, _C, _C)


def _main_body(x0t, aobs, pxc, pyc, pxr, pyr,
               wg0, bg0, wt0, bt0, wg1, bg1, wt1, bt1, wg2, bg2, wt2, bt2,
               wg3, bg3, wt3, bt3, wg4, bg4, wt4, bt4,
               out, A_s, xs, ys, zc):
    path = pl.program_id(1)
    f32 = jnp.float32
    bf = jnp.bfloat16
    N = _N

    dx = pxc[0] - pxr[0]
    dy = pyc[0] - pyr[0]
    d = jnp.sqrt(dx * dx + dy * dy + 1e-12)
    adj = d <= _GROUP_TH
    col_ids = jax.lax.broadcasted_iota(jnp.int32, (N, N), 1)
    row_ids = jax.lax.broadcasted_iota(jnp.int32, (N, N), 0)
    gc = jnp.min(jnp.where(adj, col_ids, N), axis=1, keepdims=True)
    gr = jnp.min(jnp.where(adj, row_ids, N), axis=0, keepdims=True)

    @pl.when(path == 0)
    def _():
        for t in range(_T):
            ad = aobs[0, 1, t]
            for si, s in enumerate(_SCALES):
                A_s[t, :, si * N:(si + 1) * N] = (
                    (ad <= s).astype(f32) * ad).astype(bf)

    @pl.when(path == 1)
    def _():
        same = (gc == gr).astype(f32)
        for r in range(_R):
            for t in range(_T):
                A_s[t, :, r * N:(r + 1) * N] = (
                    aobs[0, r, t] * same).astype(bf)

    @pl.when(path == 2)
    def _():
        G = (col_ids == gc).astype(bf)
        Gt = (row_ids == gr).astype(bf)
        noteye = row_ids != col_ids
        for r in range(_R):
            for t in range(_T):
                A_s[t, :, r * N:(r + 1) * N] = jnp.dot(
                    Gt, aobs[0, r, t].astype(bf),
                    preferred_element_type=f32).astype(bf)
        for r in range(_R):
            for t in range(_T):
                P2 = jnp.dot(A_s[t, :, r * N:(r + 1) * N], G,
                             preferred_element_type=f32)
                A_s[t, :, r * N:(r + 1) * N] = jnp.logical_and(
                    P2 > 0, noteye).astype(bf)
        for r in range(_R):
            for t in range(_T):
                U = jnp.dot(G, A_s[t, :, r * N:(r + 1) * N],
                            preferred_element_type=f32).astype(bf)
                A_s[t, :, r * N:(r + 1) * N] = jnp.dot(
                    U, Gt, preferred_element_type=f32).astype(bf)

    wgs = (wg0, wg1, wg2, wg3, wg4)
    bgs = (bg0, bg1, bg2, bg3, bg4)
    wts = (wt0, wt1, wt2, wt3, wt4)
    bts = (bt0, bt1, bt2, bt3, bt4)
    ys[0] = jnp.zeros((_C, N), jnp.bfloat16)
    ys[_T + 1] = jnp.zeros((_C, N), jnp.bfloat16)
    for l in range(5):
        wg, bgv, wt, btv = wgs[l], bgs[l], wts[l], bts[l]
        cin = _CINS[l]
        for t in range(_T):
            xbf = (x0t[0, t] if l == 0 else xs[t]).astype(bf)
            for r in range(_R):
                zc[r * cin:(r + 1) * cin] = jax.lax.dot_general(
                    xbf, A_s[t, :, r * N:(r + 1) * N], _DN_T,
                    preferred_element_type=f32).astype(bf)
            acc = jnp.dot(wg[...], zc[0:_R * cin],
                          preferred_element_type=f32)
            ys[t + 1] = (acc + bgv[...]).astype(bf)
        for t in range(_T):
            blk = ys[t:t + 3].reshape(3 * _C, N)
            acc2 = jnp.dot(wt[...], blk, preferred_element_type=f32) + btv[...]
            xs[t] = acc2 if l == 4 else jnp.maximum(acc2, 0.0)

    @pl.when(path == 0)
    def _():
        out[0] = xs[...]

    @pl.when(path != 0)
    def _():
        out[0] = out[0] + xs[...]


def _head_body(wtime, fused, o):
    o[0] = jnp.dot(wtime[...],
                   (fused[0] * (1.0 / 3.0)).astype(jnp.bfloat16),
                   preferred_element_type=jnp.float32)


def kernel(V_obs, A_obs, Wg0, bg0, Wt0, bt0, Wg1, bg1, Wt1, bt1, Wg2, bg2,
           Wt2, bt2, Wg3, bg3, Wt3, bt3, Wg4, bg4, Wt4, bt4, Wtime):
    f32 = jnp.float32
    bf = jnp.bfloat16
    B = V_obs.shape[0]
    p = V_obs[:, -1]
    pxc = p[:, :, 0:1]
    pyc = p[:, :, 1:2]
    pxr = p[:, :, 0][:, None, :]
    pyr = p[:, :, 1][:, None, :]
    x0t = V_obs.transpose(0, 1, 3, 2)
    Wg4p = jnp.zeros((_R, _C, _C), f32).at[:, :, :5].set(Wg4)
    Wt4p = jnp.zeros((3, _C, _C), f32).at[:, :5, :5].set(Wt4)
    bg4p = jnp.zeros((_C,), f32).at[:5].set(bg4)
    bt4p = jnp.zeros((_C,), f32).at[:5].set(bt4)

    def _prep(wg, bg, wt, bt):
        wgcat = jnp.concatenate([wg[r].T for r in range(_R)], axis=1)
        wtcat = jnp.concatenate([wt[k].T for k in range(3)], axis=1)
        return (wgcat.astype(bf), bg.reshape(-1, 1),
                wtcat.astype(bf), bt.reshape(-1, 1))

    weights = (_prep(Wg0, bg0, Wt0, bt0) + _prep(Wg1, bg1, Wt1, bt1)
               + _prep(Wg2, bg2, Wt2, bt2) + _prep(Wg3, bg3, Wt3, bt3)
               + _prep(Wg4p, bg4p, Wt4p, bt4p))

    def _full(w):
        nd = len(w.shape)
        return pl.BlockSpec(w.shape, lambda b, q, _n=nd: (0,) * _n)

    fused = pl.pallas_call(
        _main_body,
        grid=(B, 3),
        in_specs=[
            pl.BlockSpec((1, _T, 2, _N), lambda b, q: (b, 0, 0, 0)),
            pl.BlockSpec((1, _R, _T, _N, _N), lambda b, q: (b, 0, 0, 0, 0)),
            pl.BlockSpec((1, _N, 1), lambda b, q: (b, 0, 0)),
            pl.BlockSpec((1, _N, 1), lambda b, q: (b, 0, 0)),
            pl.BlockSpec((1, 1, _N), lambda b, q: (b, 0, 0)),
            pl.BlockSpec((1, 1, _N), lambda b, q: (b, 0, 0)),
        ] + [_full(w) for w in weights],
        out_specs=pl.BlockSpec((1, _T, _C, _N), lambda b, q: (b, 0, 0, 0)),
        out_shape=jax.ShapeDtypeStruct((B, _T, _C, _N), f32),
        scratch_shapes=[
            pltpu.VMEM((_T, _N, _R * _N), bf),
            pltpu.VMEM((_T, _C, _N), f32),
            pltpu.VMEM((_T + 2, _C, _N), bf),
            pltpu.VMEM((_R * _C, _N), bf),
        ],
    )(x0t, A_obs, pxc, pyc, pxr, pyr, *weights)

    fused_flat = fused.reshape(B, _T, _C * _N)
    out2 = pl.pallas_call(
        _head_body,
        grid=(B,),
        in_specs=[
            pl.BlockSpec((_PRED, _T), lambda b: (0, 0)),
            pl.BlockSpec((1, _T, _C * _N), lambda b: (b, 0, 0)),
        ],
        out_specs=pl.BlockSpec((1, _PRED, _C * _N), lambda b: (b, 0, 0)),
        out_shape=jax.ShapeDtypeStruct((B, _PRED, _C * _N), f32),
    )(Wtime.astype(bf), fused_flat)
    pred = out2.reshape(B, _PRED, _C, _N)[:, :, :5, :].transpose(0, 2, 1, 3)
    return pred


# speedup vs baseline: 61.6890x; 1.0183x over previous
import functools

import jax
import jax.numpy as jnp
from jax import lax
from jax.experimental import pallas as pl
from jax.experimental.pallas import tpu as pltpu
from jax.experimental.pallas import tpu_sc as plsc

_R = 3
_SCALES = (0.5, 1.0, 2.0)
_GROUP_TH = 2.0
_T = 8
_N = 256
_PRED = 12
_C = 64
_NSL = 2 * _R * _T

_DN_T = (((1,), (1,)), ((), ()))
_CINS = (2, _C, _C, _C, _C)


def _group_vecs(pxc, pyc, pxr, pyr):
    N = _N
    dx = pxc[0] - pxr[0]
    dy = pyc[0] - pyr[0]
    d = jnp.sqrt(dx * dx + dy * dy + 1e-12)
    adj = d <= _GROUP_TH
    col_ids = jax.lax.broadcasted_iota(jnp.int32, (N, N), 1)
    row_ids = jax.lax.broadcasted_iota(jnp.int32, (N, N), 0)
    gc = jnp.min(jnp.where(adj, col_ids, N), axis=1, keepdims=True)
    gr = jnp.min(jnp.where(adj, row_ids, N), axis=0, keepdims=True)
    return gc, gr, col_ids, row_ids


def _pool_body(aobs, pxc, pyc, pxr, pyr, sgc, gout):
    f32 = jnp.float32
    gc, gr, col_ids, row_ids = _group_vecs(pxc, pyc, pxr, pyr)
    gout[0] = gr
    G = (col_ids == gc).astype(f32)
    Gt = (row_ids == gr).astype(f32)
    noteye = row_ids != col_ids
    for r in range(_R):
        for t in range(_T):
            sgc[0, r, t] = jnp.dot(Gt, aobs[0, r, t],
                                   preferred_element_type=f32)
    for r in range(_R):
        for t in range(_T):
            P2 = jnp.dot(sgc[0, r, t], G, preferred_element_type=f32)
            sgc[0, r, t] = jnp.logical_and(P2 > 0, noteye).astype(f32)
    for r in range(_R):
        for t in range(_T):
            sgc[0, r, t] = jnp.dot(sgc[0, r, t], Gt,
                                   preferred_element_type=f32)


def _sc_gather(sgc2d, g2d):
    mesh = plsc.VectorSubcoreMesh(core_axis_name="c", subcore_axis_name="s")

    @functools.partial(
        pl.kernel,
        out_type=jax.ShapeDtypeStruct((_NSL * _N, _N), jnp.float32),
        mesh=mesh,
        scratch_types=[
            pltpu.VMEM((_N,), jnp.int32),
            pltpu.VMEM((2, 128), jnp.int32),
            pltpu.VMEM((_N, _N), jnp.float32),
            pltpu.SemaphoreType.DMA,
        ],
    )
    def k(mt_hbm, g_hbm, out_hbm, gv, idx_v, rows_v, sem):
        wid = lax.axis_index("s") * 2 + lax.axis_index("c")
        for base in (0, 32):
            s = wid + base

            @pl.when(s < _NSL)
            def _():
                b = s // (_R * _T)
                pltpu.sync_copy(g_hbm.at[b], gv)
                off = s * _N
                for c2 in range(2):
                    for c in range(8):
                        idx_v[c2, pl.ds(c * 16, 16)] = (
                            gv[pl.ds(c2 * 128 + c * 16, 16)] + off)
                for c2 in range(2):
                    pltpu.async_copy(
                        mt_hbm.at[idx_v.at[c2]],
                        rows_v.at[pl.ds(c2 * 128, 128)], sem).wait()
                pltpu.sync_copy(rows_v, out_hbm.at[pl.ds(off, _N)])

    return k(sgc2d, g2d)


def _backbone(x0t, wgs, bgs, wts, bts, A_s, xs, ys, zc):
    f32 = jnp.float32
    bf = jnp.bfloat16
    N = _N
    ys[0] = jnp.zeros((_C, N), bf)
    ys[_T + 1] = jnp.zeros((_C, N), bf)
    for l in range(5):
        wg, bgv, wt, btv = wgs[l], bgs[l], wts[l], bts[l]
        cin = _CINS[l]
        for t in range(_T):
            xbf = (x0t[0, t] if l == 0 else xs[t]).astype(bf)
            for r in range(_R):
                zc[r * cin:(r + 1) * cin] = jax.lax.dot_general(
                    xbf, A_s[t, :, r * N:(r + 1) * N], _DN_T,
                    preferred_element_type=f32).astype(bf)
            acc = jnp.dot(wg[...], zc[0:_R * cin],
                          preferred_element_type=f32)
            ys[t + 1] = (acc + bgv[...]).astype(bf)
        for t in range(_T):
            blk = ys[t:t + 3].reshape(3 * _C, N)
            acc2 = jnp.dot(wt[...], blk, preferred_element_type=f32) + btv[...]
            xs[t] = acc2 if l == 4 else jnp.maximum(acc2, 0.0)


def _mainA_body(x0t, aobs, pxc, pyc, pxr, pyr,
                wg0, bg0, wt0, bt0, wg1, bg1, wt1, bt1, wg2, bg2, wt2, bt2,
                wg3, bg3, wt3, bt3, wg4, bg4, wt4, bt4,
                out, A_s, xs, ys, zc):
    path = pl.program_id(1)
    f32 = jnp.float32
    bf = jnp.bfloat16
    N = _N
    gc, gr, col_ids, row_ids = _group_vecs(pxc, pyc, pxr, pyr)

    @pl.when(path == 0)
    def _():
        for t in range(_T):
            ad = aobs[0, 1, t]
            for si, s in enumerate(_SCALES):
                A_s[t, :, si * N:(si + 1) * N] = (
                    (ad <= s).astype(f32) * ad).astype(bf)

    @pl.when(path == 1)
    def _():
        same = (gc == gr).astype(f32)
        for r in range(_R):
            for t in range(_T):
                A_s[t, :, r * N:(r + 1) * N] = (
                    aobs[0, r, t] * same).astype(bf)

    _backbone(x0t, (wg0, wg1, wg2, wg3, wg4), (bg0, bg1, bg2, bg3, bg4),
              (wt0, wt1, wt2, wt3, wt4), (bt0, bt1, bt2, bt3, bt4),
              A_s, xs, ys, zc)

    @pl.when(path == 0)
    def _():
        out[0] = xs[...]

    @pl.when(path != 0)
    def _():
        out[0] = out[0] + xs[...]


def _mainB_body(x0t, f01, ait, pxc, pyc, pxr, pyr,
                wg0, bg0, wt0, bt0, wg1, bg1, wt1, bt1, wg2, bg2, wt2, bt2,
                wg3, bg3, wt3, bt3, wg4, bg4, wt4, bt4,
                out, A_s, xs, ys, zc):
    bf = jnp.bfloat16
    N = _N
    for r in range(_R):
        for t in range(_T):
            A_s[t, :, r * N:(r + 1) * N] = ait[0, r, t].astype(bf)

    _backbone(x0t, (wg0, wg1, wg2, wg3, wg4), (bg0, bg1, bg2, bg3, bg4),
              (wt0, wt1, wt2, wt3, wt4), (bt0, bt1, bt2, bt3, bt4),
              A_s, xs, ys, zc)

    out[0] = f01[0] + xs[...]


def _head_body(wtime, fused, o):
    o[0] = jnp.dot(wtime[...],
                   (fused[0] * (1.0 / 3.0)).astype(jnp.bfloat16),
                   preferred_element_type=jnp.float32)


def kernel(V_obs, A_obs, Wg0, bg0, Wt0, bt0, Wg1, bg1, Wt1, bt1, Wg2, bg2,
           Wt2, bt2, Wg3, bg3, Wt3, bt3, Wg4, bg4, Wt4, bt4, Wtime):
    f32 = jnp.float32
    bf = jnp.bfloat16
    B = V_obs.shape[0]
    p = V_obs[:, -1]
    pxc = p[:, :, 0:1]
    pyc = p[:, :, 1:2]
    pxr = p[:, :, 0][:, None, :]
    pyr = p[:, :, 1][:, None, :]
    x0t = V_obs.transpose(0, 1, 3, 2)

    pspec = [
        pl.BlockSpec((1, _R, _T, _N, _N), lambda b: (b, 0, 0, 0, 0)),
        pl.BlockSpec((1, _N, 1), lambda b: (b, 0, 0)),
        pl.BlockSpec((1, _N, 1), lambda b: (b, 0, 0)),
        pl.BlockSpec((1, 1, _N), lambda b: (b, 0, 0)),
        pl.BlockSpec((1, 1, _N), lambda b: (b, 0, 0)),
    ]
    sgc, g2d = pl.pallas_call(
        _pool_body,
        grid=(B,),
        in_specs=pspec,
        out_specs=[
            pl.BlockSpec((1, _R, _T, _N, _N), lambda b: (b, 0, 0, 0, 0)),
            pl.BlockSpec((1, 1, _N), lambda b: (b, 0, 0)),
        ],
        out_shape=[
            jax.ShapeDtypeStruct((B, _R, _T, _N, _N), f32),
            jax.ShapeDtypeStruct((B, 1, _N), jnp.int32),
        ],
    )(A_obs, pxc, pyc, pxr, pyr)

    ait2d = _sc_gather(sgc.reshape(_NSL * _N, _N), g2d.reshape(B, _N))
    ait = ait2d.reshape(B, _R, _T, _N, _N)

    Wg4p = jnp.zeros((_R, _C, _C), f32).at[:, :, :5].set(Wg4)
    Wt4p = jnp.zeros((3, _C, _C), f32).at[:, :5, :5].set(Wt4)
    bg4p = jnp.zeros((_C,), f32).at[:5].set(bg4)
    bt4p = jnp.zeros((_C,), f32).at[:5].set(bt4)

    def _prep(wg, bg, wt, bt):
        wgcat = jnp.concatenate([wg[r].T for r in range(_R)], axis=1)
        wtcat = jnp.concatenate([wt[k].T for k in range(3)], axis=1)
        return (wgcat.astype(bf), bg.reshape(-1, 1),
                wtcat.astype(bf), bt.reshape(-1, 1))

    weights = (_prep(Wg0, bg0, Wt0, bt0) + _prep(Wg1, bg1, Wt1, bt1)
               + _prep(Wg2, bg2, Wt2, bt2) + _prep(Wg3, bg3, Wt3, bt3)
               + _prep(Wg4p, bg4p, Wt4p, bt4p))

    def _full(w):
        nd = len(w.shape)
        return pl.BlockSpec(w.shape, lambda b, q, _n=nd: (0,) * _n)

    wspecs = [_full(w) for w in weights]
    pxspecs = [
        pl.BlockSpec((1, _N, 1), lambda b, q: (b, 0, 0)),
        pl.BlockSpec((1, _N, 1), lambda b, q: (b, 0, 0)),
        pl.BlockSpec((1, 1, _N), lambda b, q: (b, 0, 0)),
        pl.BlockSpec((1, 1, _N), lambda b, q: (b, 0, 0)),
    ]
    scratches = [
        pltpu.VMEM((_T, _N, _R * _N), bf),
        pltpu.VMEM((_T, _C, _N), f32),
        pltpu.VMEM((_T + 2, _C, _N), bf),
        pltpu.VMEM((_R * _C, _N), bf),
    ]
    fused01 = pl.pallas_call(
        _mainA_body,
        grid=(B, 2),
        in_specs=[
            pl.BlockSpec((1, _T, 2, _N), lambda b, q: (b, 0, 0, 0)),
            pl.BlockSpec((1, _R, _T, _N, _N), lambda b, q: (b, 0, 0, 0, 0)),
        ] + pxspecs + wspecs,
        out_specs=pl.BlockSpec((1, _T, _C, _N), lambda b, q: (b, 0, 0, 0)),
        out_shape=jax.ShapeDtypeStruct((B, _T, _C, _N), f32),
        scratch_shapes=scratches,
    )(x0t, A_obs, pxc, pyc, pxr, pyr, *weights)

    fused = pl.pallas_call(
        _mainB_body,
        grid=(B, 1),
        in_specs=[
            pl.BlockSpec((1, _T, 2, _N), lambda b, q: (b, 0, 0, 0)),
            pl.BlockSpec((1, _T, _C, _N), lambda b, q: (b, 0, 0, 0)),
            pl.BlockSpec((1, _R, _T, _N, _N), lambda b, q: (b, 0, 0, 0, 0)),
        ] + pxspecs + wspecs,
        out_specs=pl.BlockSpec((1, _T, _C, _N), lambda b, q: (b, 0, 0, 0)),
        out_shape=jax.ShapeDtypeStruct((B, _T, _C, _N), f32),
        scratch_shapes=scratches,
    )(x0t, fused01, ait, pxc, pyc, pxr, pyr, *weights)

    fused_flat = fused.reshape(B, _T, _C * _N)
    out2 = pl.pallas_call(
        _head_body,
        grid=(B,),
        in_specs=[
            pl.BlockSpec((_PRED, _T), lambda b: (0, 0)),
            pl.BlockSpec((1, _T, _C * _N), lambda b: (b, 0, 0)),
        ],
        out_specs=pl.BlockSpec((1, _PRED, _C * _N), lambda b: (b, 0, 0)),
        out_shape=jax.ShapeDtypeStruct((B, _PRED, _C * _N), f32),
    )(Wtime.astype(bf), fused_flat)
    pred = out2.reshape(B, _PRED, _C, _N)[:, :, :5, :].transpose(0, 2, 1, 3)
    return pred
